# Initial kernel scaffold; baseline (speedup 1.0000x reference)
import jax
import jax.numpy as jnp
from jax.experimental import pallas as pl


def kernel(feature, edge_index, snorm_n, W):
    raise NotImplementedError("write your pallas kernel here")


# speedup vs baseline: 5.9863x; 5.9863x over previous
import functools

import jax
import jax.numpy as jnp
from jax import lax
from jax.experimental import pallas as pl
from jax.experimental.pallas import tpu as pltpu
from jax.experimental.pallas import tpu_sc as plsc

N_NODES = 10000
N_PAD = 10240
N_EDGES = 320000
D = 128

NC = 2
NS = 16
NW = NC * NS
E_PER_W = N_EDGES // NW
CHUNK = 80
N_CHUNKS = E_PER_W // CHUNK
ROWS_PER_S = N_PAD // NS


def _sc_body(feat_hbm, src_hbm, dst_hbm, agg_out, deg_out,
             src_v, dst_v, rows_v, ones_v, zrow_v, degbuf_v, sem,
             agg_sh, deg_sh):
    c = lax.axis_index("c")
    s = lax.axis_index("s")
    wid = s * NC + c

    zeros16 = jnp.zeros((16,), jnp.float32)
    ones16 = jnp.ones((16,), jnp.float32)
    for i in range(CHUNK // 16):
        ones_v[pl.ds(i * 16, 16)] = ones16

    def zrow_body(r, carry):
        for j in range(D // 16):
            zrow_v[r, pl.ds(j * 16, 16)] = zeros16
        return carry

    lax.fori_loop(0, 128, zrow_body, 0)
    for i in range(ROWS_PER_S // 16):
        degbuf_v[pl.ds(i * 16, 16)] = zeros16

    for k in range(ROWS_PER_S // 128):
        pltpu.sync_copy(zrow_v, agg_sh.at[pl.ds(s * ROWS_PER_S + k * 128, 128)])
    pltpu.sync_copy(degbuf_v, deg_sh.at[pl.ds(s * ROWS_PER_S, ROWS_PER_S)])
    plsc.subcore_barrier()

    def chunk_body(i, carry):
        base = wid * E_PER_W + i * CHUNK
        pltpu.sync_copy(src_hbm.at[pl.ds(base, CHUNK)], src_v)
        pltpu.sync_copy(dst_hbm.at[pl.ds(base, CHUNK)], dst_v)
        pltpu.async_copy(feat_hbm.at[src_v], rows_v, sem).wait()
        pltpu.sync_copy(rows_v, agg_sh.at[dst_v], add=True)
        pltpu.sync_copy(ones_v, deg_sh.at[dst_v], add=True)
        return carry

    lax.fori_loop(0, N_CHUNKS, chunk_body, 0)
    plsc.subcore_barrier()

    for k in range(ROWS_PER_S // 128):
        r0 = s * ROWS_PER_S + k * 128
        pltpu.sync_copy(agg_sh.at[pl.ds(r0, 128)], zrow_v)
        pltpu.sync_copy(zrow_v, agg_out.at[pl.ds(c * N_PAD + r0, 128)])
    pltpu.sync_copy(deg_sh.at[pl.ds(s * ROWS_PER_S, ROWS_PER_S)], degbuf_v)
    pltpu.sync_copy(degbuf_v, deg_out.at[pl.ds(c * N_PAD + s * ROWS_PER_S, ROWS_PER_S)])


_sc_scatter = pl.kernel(
    _sc_body,
    out_type=[
        jax.ShapeDtypeStruct((NC * N_PAD, D), jnp.float32),
        jax.ShapeDtypeStruct((NC * N_PAD,), jnp.float32),
    ],
    mesh=plsc.VectorSubcoreMesh(core_axis_name="c", subcore_axis_name="s"),
    scratch_types=[
        pltpu.VMEM((CHUNK,), jnp.int32),
        pltpu.VMEM((CHUNK,), jnp.int32),
        pltpu.VMEM((CHUNK, D), jnp.float32),
        pltpu.VMEM((CHUNK,), jnp.float32),
        pltpu.VMEM((128, D), jnp.float32),
        pltpu.VMEM((ROWS_PER_S,), jnp.float32),
        pltpu.SemaphoreType.DMA,
        pltpu.VMEM_SHARED((N_PAD, D), jnp.float32),
        pltpu.VMEM_SHARED((N_PAD,), jnp.float32),
    ],
)


def _tc_body(a0, a1, d0, d1, f, sn, w, out):
    agg = a0[...] + a1[...]
    deg = d0[...] + d1[...]
    mean = agg / jnp.maximum(deg, 1.0)
    h = jnp.where(deg > 0.0, mean, f[...])
    h = jnp.dot(h, w[...], preferred_element_type=jnp.float32)
    h = h * sn[...]
    out[...] = jnp.maximum(h, 0.0)


_BLK = 1024


def _tc_combine(a0, a1, d0, d1, f_pad, sn_pad, W):
    grid = (N_PAD // _BLK,)
    row_spec = pl.BlockSpec((_BLK, D), lambda i: (i, 0))
    col_spec = pl.BlockSpec((_BLK, 1), lambda i: (i, 0))
    w_spec = pl.BlockSpec((D, D), lambda i: (0, 0))
    return pl.pallas_call(
        _tc_body,
        grid=grid,
        in_specs=[row_spec, row_spec, col_spec, col_spec, row_spec, col_spec, w_spec],
        out_specs=row_spec,
        out_shape=jax.ShapeDtypeStruct((N_PAD, D), jnp.float32),
    )(a0, a1, d0, d1, f_pad, sn_pad, W)


@jax.jit
def kernel(feature, edge_index, snorm_n, W):
    src = edge_index[0]
    dst = edge_index[1]
    agg2, deg2 = _sc_scatter(feature, src, dst)
    a0 = agg2[:N_PAD]
    a1 = agg2[N_PAD:]
    d0 = deg2[:N_PAD].reshape(N_PAD, 1)
    d1 = deg2[N_PAD:].reshape(N_PAD, 1)
    f_pad = jnp.pad(feature, ((0, N_PAD - N_NODES), (0, 0)))
    sn_pad = jnp.pad(snorm_n, ((0, N_PAD - N_NODES), (0, 0)))
    h = _tc_combine(a0, a1, d0, d1, f_pad, sn_pad, W)
    return h[:N_NODES]


# speedup vs baseline: 12.2337x; 2.0436x over previous
import jax
import jax.numpy as jnp
from jax import lax
from jax.experimental import pallas as pl
from jax.experimental.pallas import tpu as pltpu
from jax.experimental.pallas import tpu_sc as plsc

N_NODES = 10000
N_PAD = 10240
N_EDGES = 320000
D = 128

NC = 2
NS = 16
NW = NC * NS
CHUNK = 128
E_MAIN = 10240
E_LAST = N_EDGES - E_MAIN * (NW - 1)
ROWS_MAIN = E_MAIN // CHUNK
ROWS_LAST = E_LAST // CHUNK
ROWS_PER_S = N_PAD // NS


def _sc_body(feat_hbm, src_hbm, dst_hbm, agg_out, deg_out,
             srcc0, srcc1, dstc0, dstc1, rows0, rows1, ones_v, degbuf_v,
             semg0, semg1, semi0, semi1, agg_sh, deg_sh):
    c = lax.axis_index("c")
    s = lax.axis_index("s")
    wid = s * NC + c
    base = wid * E_MAIN
    nrows = jnp.where(wid == NW - 1, ROWS_LAST, ROWS_MAIN)

    zeros16 = jnp.zeros((16,), jnp.float32)
    ones16 = jnp.ones((16,), jnp.float32)
    for i in range(CHUNK // 16):
        ones_v[pl.ds(i * 16, 16)] = ones16

    def zrow_body(r, carry):
        for j in range(D // 16):
            rows0[r, pl.ds(j * 16, 16)] = zeros16
        return carry

    lax.fori_loop(0, 128, zrow_body, 0)
    for i in range(ROWS_PER_S // 16):
        degbuf_v[pl.ds(i * 16, 16)] = zeros16

    for k in range(ROWS_PER_S // 128):
        pltpu.sync_copy(rows0, agg_sh.at[pl.ds(s * ROWS_PER_S + k * 128, 128)])
    pltpu.sync_copy(degbuf_v, deg_sh.at[pl.ds(s * ROWS_PER_S, ROWS_PER_S)])

    pltpu.sync_copy(src_hbm.at[pl.ds(base, CHUNK)], srcc0)
    pltpu.sync_copy(dst_hbm.at[pl.ds(base, CHUNK)], dstc0)
    pltpu.async_copy(src_hbm.at[pl.ds(base + CHUNK, CHUNK)], srcc1, semi1)
    pltpu.async_copy(dst_hbm.at[pl.ds(base + CHUNK, CHUNK)], dstc1, semi1)
    pltpu.async_copy(feat_hbm.at[srcc0], rows0, semg0)
    plsc.subcore_barrier()

    bufs = ((srcc0, dstc0, rows0, semg0, semi0),
            (srcc1, dstc1, rows1, semg1, semi1))

    def chunk_body(ko, carry):
        for b in range(2):
            i = 2 * ko + b
            srcc, dstc, rows, semg, semi = bufs[b]
            srcco, dstco, rowso, semgo, semio = bufs[1 - b]

            pltpu.make_async_copy(feat_hbm.at[srcc], rows, semg).wait()

            @pl.when(i + 1 < nrows)
            def _():
                pltpu.make_async_copy(
                    src_hbm.at[pl.ds(base + (i + 1) * CHUNK, CHUNK)], srcco,
                    semio).wait()
                pltpu.make_async_copy(
                    dst_hbm.at[pl.ds(base + (i + 1) * CHUNK, CHUNK)], dstco,
                    semio).wait()
                pltpu.async_copy(feat_hbm.at[srcco], rowso, semgo)

            pltpu.sync_copy(rows, agg_sh.at[dstc], add=True)
            pltpu.sync_copy(ones_v, deg_sh.at[dstc], add=True)

            @pl.when(i + 2 < nrows)
            def _():
                pltpu.async_copy(
                    src_hbm.at[pl.ds(base + (i + 2) * CHUNK, CHUNK)], srcc, semi)
                pltpu.async_copy(
                    dst_hbm.at[pl.ds(base + (i + 2) * CHUNK, CHUNK)], dstc, semi)

        return carry

    lax.fori_loop(0, nrows // 2, chunk_body, 0)
    plsc.subcore_barrier()

    for k in range(ROWS_PER_S // 128):
        r0 = s * ROWS_PER_S + k * 128
        pltpu.sync_copy(agg_sh.at[pl.ds(r0, 128)], rows0)
        pltpu.sync_copy(rows0, agg_out.at[pl.ds(c * N_PAD + r0, 128)])
    pltpu.sync_copy(deg_sh.at[pl.ds(s * ROWS_PER_S, ROWS_PER_S)], degbuf_v)
    pltpu.sync_copy(degbuf_v, deg_out.at[pl.ds(c * N_PAD + s * ROWS_PER_S, ROWS_PER_S)])


_sc_scatter = pl.kernel(
    _sc_body,
    out_type=[
        jax.ShapeDtypeStruct((NC * N_PAD, D), jnp.float32),
        jax.ShapeDtypeStruct((NC * N_PAD,), jnp.float32),
    ],
    mesh=plsc.VectorSubcoreMesh(core_axis_name="c", subcore_axis_name="s"),
    scratch_types=[
        pltpu.VMEM((CHUNK,), jnp.int32),
        pltpu.VMEM((CHUNK,), jnp.int32),
        pltpu.VMEM((CHUNK,), jnp.int32),
        pltpu.VMEM((CHUNK,), jnp.int32),
        pltpu.VMEM((CHUNK, D), jnp.float32),
        pltpu.VMEM((CHUNK, D), jnp.float32),
        pltpu.VMEM((CHUNK,), jnp.float32),
        pltpu.VMEM((ROWS_PER_S,), jnp.float32),
        pltpu.SemaphoreType.DMA,
        pltpu.SemaphoreType.DMA,
        pltpu.SemaphoreType.DMA,
        pltpu.SemaphoreType.DMA,
        pltpu.VMEM_SHARED((N_PAD, D), jnp.float32),
        pltpu.VMEM_SHARED((N_PAD,), jnp.float32),
    ],
)


def _tc_body(agg2, deg2, f, sn, w, out):
    a = agg2[...]
    d = deg2[...]
    agg = a[0] + a[1]
    deg = d[0] + d[1]
    mean = agg / jnp.maximum(deg, 1.0)
    h = jnp.where(deg > 0.0, mean, f[...])
    h = jnp.dot(h, w[...], preferred_element_type=jnp.float32)
    h = h * sn[...]
    out[...] = jnp.maximum(h, 0.0)


_BLK = 1000


def _tc_combine(agg2, deg2, feature, snorm_n, W):
    grid = (N_NODES // _BLK,)
    return pl.pallas_call(
        _tc_body,
        grid=grid,
        in_specs=[
            pl.BlockSpec((NC, _BLK, D), lambda i: (0, i, 0)),
            pl.BlockSpec((NC, _BLK, 1), lambda i: (0, i, 0)),
            pl.BlockSpec((_BLK, D), lambda i: (i, 0)),
            pl.BlockSpec((_BLK, 1), lambda i: (i, 0)),
            pl.BlockSpec((D, D), lambda i: (0, 0)),
        ],
        out_specs=pl.BlockSpec((_BLK, D), lambda i: (i, 0)),
        out_shape=jax.ShapeDtypeStruct((N_NODES, D), jnp.float32),
    )(agg2, deg2, feature, snorm_n, W)


@jax.jit
def kernel(feature, edge_index, snorm_n, W):
    src = edge_index[0]
    dst = edge_index[1]
    agg2, deg2 = _sc_scatter(feature, src, dst)
    return _tc_combine(agg2.reshape(NC, N_PAD, D), deg2.reshape(NC, N_PAD, 1),
                       feature, snorm_n, W)
